# Initial kernel scaffold; baseline (speedup 1.0000x reference)
import jax
import jax.numpy as jnp
from jax.experimental import pallas as pl


def kernel(x, pos_table):
    raise NotImplementedError("write your pallas kernel here")


# speedup vs baseline: 2.8300x; 2.8300x over previous
import jax
import jax.numpy as jnp
from jax.experimental import pallas as pl


S_BLK = 512


def _add_pe_kernel(x_ref, pe_ref, o_ref):
    o_ref[...] = x_ref[...] + pe_ref[...]


def kernel(x, pos_table):
    b, s, d = x.shape
    grid = (s // S_BLK, b)
    return pl.pallas_call(
        _add_pe_kernel,
        grid=grid,
        in_specs=[
            pl.BlockSpec((1, S_BLK, d), lambda i, j: (j, i, 0)),
            pl.BlockSpec((S_BLK, d), lambda i, j: (i, 0)),
        ],
        out_specs=pl.BlockSpec((1, S_BLK, d), lambda i, j: (j, i, 0)),
        out_shape=jax.ShapeDtypeStruct((b, s, d), x.dtype),
    )(x, pos_table)


# speedup vs baseline: 3.1562x; 1.1153x over previous
import jax
import jax.numpy as jnp
from jax.experimental import pallas as pl


S_BLK = 1024


def _add_pe_kernel(x_ref, pe_ref, o_ref):
    o_ref[...] = x_ref[...] + pe_ref[...]


def kernel(x, pos_table):
    b, s, d = x.shape
    grid = (s // S_BLK, b)
    return pl.pallas_call(
        _add_pe_kernel,
        grid=grid,
        in_specs=[
            pl.BlockSpec((1, S_BLK, d), lambda i, j: (j, i, 0)),
            pl.BlockSpec((S_BLK, d), lambda i, j: (i, 0)),
        ],
        out_specs=pl.BlockSpec((1, S_BLK, d), lambda i, j: (j, i, 0)),
        out_shape=jax.ShapeDtypeStruct((b, s, d), x.dtype),
    )(x, pos_table)


# speedup vs baseline: 3.3380x; 1.0576x over previous
import jax
import jax.numpy as jnp
from jax.experimental import pallas as pl


S_BLK = 2048


def _add_pe_kernel(x_ref, pe_ref, o_ref):
    o_ref[...] = x_ref[...] + pe_ref[...]


def kernel(x, pos_table):
    b, s, d = x.shape
    grid = (s // S_BLK, b)
    return pl.pallas_call(
        _add_pe_kernel,
        grid=grid,
        in_specs=[
            pl.BlockSpec((1, S_BLK, d), lambda i, j: (j, i, 0)),
            pl.BlockSpec((S_BLK, d), lambda i, j: (i, 0)),
        ],
        out_specs=pl.BlockSpec((1, S_BLK, d), lambda i, j: (j, i, 0)),
        out_shape=jax.ShapeDtypeStruct((b, s, d), x.dtype),
    )(x, pos_table)
